# baseline (speedup 1.0000x reference)
import functools

import jax
import jax.numpy as jnp
from jax import lax
from jax.experimental import pallas as pl
from jax.experimental.pallas import tpu as pltpu
from jax.experimental.pallas import tpu_sc as plsc

_VOCAB = 1000000
_D = 64
_DP = 128
_SEQ = 200
_BATCH = 1024

_NC = 2
_NS = 16
_NW = _NC * _NS
_SEQ_PER_W = _BATCH // _NW
_ROWS_PER_W = _SEQ_PER_W * _SEQ
_CSEQ = 2
_CROWS = _CSEQ * _SEQ
_NCHUNK = _SEQ_PER_W // _CSEQ


def _build_sc_call():
    mesh = plsc.VectorSubcoreMesh(core_axis_name="c", subcore_axis_name="s")

    @functools.partial(
        pl.kernel,
        mesh=mesh,
        compiler_params=pltpu.CompilerParams(use_tc_tiling_on_sc=False),
        out_type=jax.ShapeDtypeStruct((_SEQ, _BATCH, _DP), jnp.float32),
        scratch_types=[
            pltpu.VMEM((_SEQ_PER_W, _SEQ), jnp.int32),
            pltpu.VMEM((_ROWS_PER_W,), jnp.int32),
            pltpu.VMEM((2, _CROWS, _DP), jnp.float32),
            pltpu.SemaphoreType.DMA,
            pltpu.SemaphoreType.DMA,
            pltpu.SemaphoreType.DMA,
            pltpu.SemaphoreType.DMA,
            pltpu.SemaphoreType.DMA,
        ],
    )
    def sc_gather(ids_hbm, remap_hbm, emb_hbm, out_hbm,
                  ids_v, dict_v, rows_v,
                  sem_r, sem_e0, sem_e1, sem_o0, sem_o1):
        wid = lax.axis_index("s") * _NC + lax.axis_index("c")
        seq0 = wid * _SEQ_PER_W
        row0 = wid * _ROWS_PER_W
        sem_e = (sem_e0, sem_e1)
        sem_o = (sem_o0, sem_o1)

        pltpu.sync_copy(ids_hbm.at[pl.ds(seq0, _SEQ_PER_W)], ids_v)

        def remap_row(j):
            return pltpu.make_async_copy(
                remap_hbm.at[ids_v.at[j]],
                dict_v.at[pl.ds(j * _SEQ, _SEQ)],
                sem_r,
            )

        def fire_remap(j, carry):
            remap_row(j).start()
            return carry

        def drain_remap(j, carry):
            remap_row(j).wait()
            return carry

        lax.fori_loop(0, _SEQ_PER_W, fire_remap, 0)
        lax.fori_loop(0, _SEQ_PER_W, drain_remap, 0)

        def dict_slice(i):
            return dict_v.at[pl.ds(i * _CROWS, _CROWS)]

        def start_emb(i, b):
            pltpu.make_async_copy(
                emb_hbm.at[dict_slice(i)], rows_v.at[b], sem_e[b],
            ).start()

        def wait_emb(i, b):
            pltpu.make_async_copy(
                emb_hbm.at[dict_slice(i)], rows_v.at[b], sem_e[b],
            ).wait()

        def out_copies(i, b):
            bcol = seq0 + i * _CSEQ
            return [
                pltpu.make_async_copy(
                    rows_v.at[b].at[pl.ds(c * _SEQ, _SEQ)],
                    out_hbm.at[:, bcol + c],
                    sem_o[b],
                )
                for c in range(_CSEQ)
            ]

        start_emb(0, 0)

        def step(i, b):
            wait_emb(i, b)

            @pl.when(i + 1 < _NCHUNK)
            def _():
                @pl.when(i >= 1)
                def _():
                    for cp in out_copies(i - 1, 1 - b):
                        cp.wait()
                start_emb(i + 1, 1 - b)

            for cp in out_copies(i, b):
                cp.start()

        def pair(g, carry):
            step(2 * g, 0)
            step(2 * g + 1, 1)
            return carry

        lax.fori_loop(0, _NCHUNK // 2, pair, 0)

        for cp in out_copies(_NCHUNK - 2, 0):
            cp.wait()
        for cp in out_copies(_NCHUNK - 1, 1):
            cp.wait()

    return sc_gather


_SC_CALL = _build_sc_call()

def _add_wpe_transpose(g_s, wpe):
    def body(g_ref, wpe_ref, eye_ref, out_ref):
        s = pl.program_id(0)
        x = g_ref[...][0][:, 0:_D]
        w = wpe_ref[s, :].reshape(_D, 1)
        xt = jax.lax.dot_general(
            eye_ref[...], x, (((1,), (1,)), ((), ())),
            precision=jax.lax.Precision.HIGHEST,
            preferred_element_type=jnp.float32)
        out_ref[...] = (xt + w)[None]

    return pl.pallas_call(
        body,
        grid=(_SEQ,),
        in_specs=[
            pl.BlockSpec((1, _BATCH, _DP), lambda s: (s, 0, 0)),
            pl.BlockSpec((_SEQ, _D), lambda s: (0, 0)),
            pl.BlockSpec((_D, _D), lambda s: (0, 0)),
        ],
        out_specs=pl.BlockSpec((1, _D, _BATCH), lambda s: (s, 0, 0)),
        out_shape=jax.ShapeDtypeStruct((_SEQ, _D, _BATCH), jnp.float32),
    )(g_s, wpe, jnp.eye(_D, dtype=jnp.float32))


def kernel(input_ids, attention_mask, embedding_dict, input_ids2dict_ids, wpe):
    emb_p = jnp.pad(embedding_dict, ((0, 0), (0, _DP - _D)))
    g_s = _SC_CALL(input_ids, input_ids2dict_ids, emb_p)
    out_t = _add_wpe_transpose(g_s, wpe)
    return jnp.transpose(out_t, (2, 0, 1)), attention_mask


# speedup vs baseline: 1.0559x; 1.0559x over previous
import functools

import jax
import jax.numpy as jnp
from jax import lax
from jax.experimental import pallas as pl
from jax.experimental.pallas import tpu as pltpu
from jax.experimental.pallas import tpu_sc as plsc

_VOCAB = 1000000
_D = 64
_DP = 128
_SEQ = 200
_BATCH = 1024

_NC = 2
_NS = 16
_NW = _NC * _NS
_SEQ_PER_W = _BATCH // _NW
_ROWS_PER_W = _SEQ_PER_W * _SEQ
_CSEQ = 2
_CROWS = _CSEQ * _SEQ
_NCHUNK = _SEQ_PER_W // _CSEQ


def _build_sc_call():
    mesh = plsc.VectorSubcoreMesh(core_axis_name="c", subcore_axis_name="s")

    @functools.partial(
        pl.kernel,
        mesh=mesh,
        compiler_params=pltpu.CompilerParams(use_tc_tiling_on_sc=False),
        out_type=jax.ShapeDtypeStruct((_SEQ, _BATCH, _DP), jnp.float32),
        scratch_types=[
            pltpu.VMEM((_SEQ_PER_W, _SEQ), jnp.int32),
            pltpu.VMEM((_ROWS_PER_W,), jnp.int32),
            pltpu.VMEM((2, _CROWS, _DP), jnp.float32),
            pltpu.SemaphoreType.DMA,
            pltpu.SemaphoreType.DMA,
            pltpu.SemaphoreType.DMA,
            pltpu.SemaphoreType.DMA,
            pltpu.SemaphoreType.DMA,
        ],
    )
    def sc_gather(ids_hbm, remap_hbm, emb_hbm, out_hbm,
                  ids_v, dict_v, rows_v,
                  sem_r, sem_e0, sem_e1, sem_o0, sem_o1):
        wid = lax.axis_index("s") * _NC + lax.axis_index("c")
        seq0 = wid * _SEQ_PER_W
        row0 = wid * _ROWS_PER_W
        sem_e = (sem_e0, sem_e1)
        sem_o = (sem_o0, sem_o1)

        pltpu.sync_copy(ids_hbm.at[pl.ds(seq0, _SEQ_PER_W)], ids_v)

        def remap_row(j):
            return pltpu.make_async_copy(
                remap_hbm.at[ids_v.at[j]],
                dict_v.at[pl.ds(j * _SEQ, _SEQ)],
                sem_r,
            )

        def fire_remap(j, carry):
            remap_row(j).start()
            return carry

        def drain_remap(j, carry):
            remap_row(j).wait()
            return carry

        lax.fori_loop(0, _SEQ_PER_W, fire_remap, 0)
        lax.fori_loop(0, _SEQ_PER_W, drain_remap, 0)

        def dict_slice(i):
            return dict_v.at[pl.ds(i * _CROWS, _CROWS)]

        def start_emb(i, b):
            pltpu.make_async_copy(
                emb_hbm.at[dict_slice(i)], rows_v.at[b], sem_e[b],
            ).start()

        def wait_emb(i, b):
            pltpu.make_async_copy(
                emb_hbm.at[dict_slice(i)], rows_v.at[b], sem_e[b],
            ).wait()

        def out_copies(i, b):
            bcol = seq0 + i * _CSEQ
            return [
                pltpu.make_async_copy(
                    rows_v.at[b].at[pl.ds(c * _SEQ, _SEQ)],
                    out_hbm.at[:, bcol + c],
                    sem_o[b],
                )
                for c in range(_CSEQ)
            ]

        start_emb(0, 0)

        def step(i, b):
            wait_emb(i, b)

            @pl.when(i + 1 < _NCHUNK)
            def _():
                @pl.when(i >= 1)
                def _():
                    for cp in out_copies(i - 1, 1 - b):
                        cp.wait()
                start_emb(i + 1, 1 - b)

            for cp in out_copies(i, b):
                cp.start()

        def pair(g, carry):
            step(2 * g, 0)
            step(2 * g + 1, 1)
            return carry

        lax.fori_loop(0, _NCHUNK // 2, pair, 0)

        for cp in out_copies(_NCHUNK - 2, 0):
            cp.wait()
        for cp in out_copies(_NCHUNK - 1, 1):
            cp.wait()

    return sc_gather


_SC_CALL = _build_sc_call()

def _add_wpe_transpose(g_s, wpe):
    def body(g_ref, wpe_ref, eye_ref, out_ref):
        s = pl.program_id(0)
        x = g_ref[...][0][:, 0:_D]
        w = wpe_ref[s, :].reshape(_D, 1)
        xt = jax.lax.dot_general(
            eye_ref[...], x, (((1,), (1,)), ((), ())),
            preferred_element_type=jnp.float32)
        out_ref[...] = (xt + w)[None]

    return pl.pallas_call(
        body,
        grid=(_SEQ,),
        in_specs=[
            pl.BlockSpec((1, _BATCH, _DP), lambda s: (s, 0, 0)),
            pl.BlockSpec((_SEQ, _D), lambda s: (0, 0)),
            pl.BlockSpec((_D, _D), lambda s: (0, 0)),
        ],
        out_specs=pl.BlockSpec((1, _D, _BATCH), lambda s: (s, 0, 0)),
        out_shape=jax.ShapeDtypeStruct((_SEQ, _D, _BATCH), jnp.float32),
    )(g_s, wpe, jnp.eye(_D, dtype=jnp.float32))


def kernel(input_ids, attention_mask, embedding_dict, input_ids2dict_ids, wpe):
    emb_p = jnp.pad(embedding_dict, ((0, 0), (0, _DP - _D)))
    g_s = _SC_CALL(input_ids, input_ids2dict_ids, emb_p)
    out_t = _add_wpe_transpose(g_s, wpe)
    return jnp.transpose(out_t, (2, 0, 1)), attention_mask


# speedup vs baseline: 1.1972x; 1.1338x over previous
import functools

import jax
import jax.numpy as jnp
from jax import lax
from jax.experimental import pallas as pl
from jax.experimental.pallas import tpu as pltpu
from jax.experimental.pallas import tpu_sc as plsc

_VOCAB = 1000000
_D = 64
_DP = 128
_SEQ = 200
_BATCH = 1024

_NC = 2
_NS = 16
_NW = _NC * _NS
_SEQ_PER_W = _BATCH // _NW
_ROWS_PER_W = _SEQ_PER_W * _SEQ
_CSEQ = 2
_CROWS = _CSEQ * _SEQ
_NCHUNK = _SEQ_PER_W // _CSEQ


def _build_sc_call():
    mesh = plsc.VectorSubcoreMesh(core_axis_name="c", subcore_axis_name="s")

    @functools.partial(
        pl.kernel,
        mesh=mesh,
        compiler_params=pltpu.CompilerParams(use_tc_tiling_on_sc=False),
        out_type=jax.ShapeDtypeStruct((_SEQ, _BATCH, _DP), jnp.float32),
        scratch_types=[
            pltpu.VMEM((_SEQ_PER_W, _SEQ), jnp.int32),
            pltpu.VMEM((_ROWS_PER_W,), jnp.int32),
            pltpu.VMEM((2, _CROWS, _DP), jnp.float32),
            pltpu.SemaphoreType.DMA,
            pltpu.SemaphoreType.DMA,
            pltpu.SemaphoreType.DMA,
            pltpu.SemaphoreType.DMA,
            pltpu.SemaphoreType.DMA,
        ],
    )
    def sc_gather(ids_hbm, remap_hbm, emb_hbm, out_hbm,
                  ids_v, dict_v, rows_v,
                  sem_r, sem_e0, sem_e1, sem_o0, sem_o1):
        wid = lax.axis_index("s") * _NC + lax.axis_index("c")
        seq0 = wid * _SEQ_PER_W
        row0 = wid * _ROWS_PER_W
        sem_e = (sem_e0, sem_e1)
        sem_o = (sem_o0, sem_o1)

        pltpu.sync_copy(ids_hbm.at[pl.ds(seq0, _SEQ_PER_W)], ids_v)

        def remap_row(j):
            return pltpu.make_async_copy(
                remap_hbm.at[ids_v.at[j]],
                dict_v.at[pl.ds(j * _SEQ, _SEQ)],
                sem_r,
            )

        def fire_remap(j, carry):
            remap_row(j).start()
            return carry

        def drain_remap(j, carry):
            remap_row(j).wait()
            return carry

        lax.fori_loop(0, _SEQ_PER_W, fire_remap, 0)
        lax.fori_loop(0, _SEQ_PER_W, drain_remap, 0)

        def dict_slice(i):
            return dict_v.at[pl.ds(i * _CROWS, _CROWS)]

        def start_emb(i, b):
            pltpu.make_async_copy(
                emb_hbm.at[dict_slice(i)], rows_v.at[b], sem_e[b],
            ).start()

        def wait_emb(i, b):
            pltpu.make_async_copy(
                emb_hbm.at[dict_slice(i)], rows_v.at[b], sem_e[b],
            ).wait()

        def out_copies(i, b):
            bcol = seq0 + i * _CSEQ
            return [
                pltpu.make_async_copy(
                    rows_v.at[b].at[pl.ds(c * _SEQ, _SEQ)],
                    out_hbm.at[:, bcol + c],
                    sem_o[b],
                )
                for c in range(_CSEQ)
            ]

        start_emb(0, 0)

        def step(i, b):
            wait_emb(i, b)

            @pl.when(i + 1 < _NCHUNK)
            def _():
                @pl.when(i >= 1)
                def _():
                    for cp in out_copies(i - 1, 1 - b):
                        cp.wait()
                start_emb(i + 1, 1 - b)

            for cp in out_copies(i, b):
                cp.start()

        def pair(g, carry):
            step(2 * g, 0)
            step(2 * g + 1, 1)
            return carry

        lax.fori_loop(0, _NCHUNK // 2, pair, 0)

        for cp in out_copies(_NCHUNK - 2, 0):
            cp.wait()
        for cp in out_copies(_NCHUNK - 1, 1):
            cp.wait()

    return sc_gather


_SC_CALL = _build_sc_call()

_SBLK = 8


def _add_wpe_transpose(g_s, wpe):
    def body(g_ref, wpe_ref, eye_ref, out_ref):
        s0 = pl.program_id(0) * _SBLK
        for c in range(_SBLK):
            x = g_ref[c][:, 0:_D]
            w = wpe_ref[s0 + c, :].reshape(_D, 1)
            xt = jax.lax.dot_general(
                eye_ref[...], x, (((1,), (1,)), ((), ())),
                preferred_element_type=jnp.float32)
            out_ref[c, :, :] = xt + w

    return pl.pallas_call(
        body,
        grid=(_SEQ // _SBLK,),
        in_specs=[
            pl.BlockSpec((_SBLK, _BATCH, _DP), lambda s: (s, 0, 0)),
            pl.BlockSpec((_SEQ, _D), lambda s: (0, 0)),
            pl.BlockSpec((_D, _D), lambda s: (0, 0)),
        ],
        out_specs=pl.BlockSpec((_SBLK, _D, _BATCH), lambda s: (s, 0, 0)),
        out_shape=jax.ShapeDtypeStruct((_SEQ, _D, _BATCH), jnp.float32),
    )(g_s, wpe, jnp.eye(_D, dtype=jnp.float32))


def kernel(input_ids, attention_mask, embedding_dict, input_ids2dict_ids, wpe):
    emb_p = jnp.pad(embedding_dict, ((0, 0), (0, _DP - _D)))
    g_s = _SC_CALL(input_ids, input_ids2dict_ids, emb_p)
    out_t = _add_wpe_transpose(g_s, wpe)
    return jnp.transpose(out_t, (2, 0, 1)), attention_mask
